# baseline (speedup 1.0000x reference)
import functools

import jax
import jax.numpy as jnp
from jax import lax
from jax.experimental import pallas as pl
from jax.experimental.pallas import tpu as pltpu
from jax.experimental.pallas import tpu_sc as plsc

_VOCAB = 1000000
_DIM = 64
_BATCH = 4096
_HIST = 50
_N = _BATCH * _HIST
_NW = 32
_PER_W = _N // _NW
_CHUNK = 640
_NCHUNK = _PER_W // _CHUNK


def _make_sc_kernel():
    mesh = plsc.VectorSubcoreMesh(core_axis_name="c", subcore_axis_name="s")

    @functools.partial(
        pl.kernel,
        mesh=mesh,
        compiler_params=pltpu.CompilerParams(use_tc_tiling_on_sc=False),
        out_type=jax.ShapeDtypeStruct((_N, _DIM), jnp.float32),
        scratch_types=[
            pltpu.VMEM((_CHUNK,), jnp.int32),
            pltpu.VMEM((_CHUNK,), jnp.int32),
            pltpu.VMEM((_CHUNK, _DIM), jnp.float32),
            pltpu.SemaphoreType.DMA,
        ],
    )
    def tok_emb(x_hbm, table_hbm, idmap_hbm, out_hbm, xv, mv, rows, sem):
        wid = lax.axis_index("s") * 2 + lax.axis_index("c")
        base = wid * _PER_W

        def body(i, carry):
            off = base + i * _CHUNK
            pltpu.sync_copy(x_hbm.at[pl.ds(off, _CHUNK)], xv)
            pltpu.async_copy(idmap_hbm.at[xv], mv, sem).wait()
            pltpu.async_copy(table_hbm.at[mv], rows, sem).wait()
            pltpu.sync_copy(rows, out_hbm.at[pl.ds(off, _CHUNK)])
            return carry

        lax.fori_loop(0, _NCHUNK, body, 0)

    return tok_emb


_SC_KERNEL = _make_sc_kernel()


def kernel(x, table, id_mapper):
    x_flat = x.reshape(_N)
    out = _SC_KERNEL(x_flat, table, id_mapper)
    return out.reshape(_BATCH, _HIST, _DIM)


# speedup vs baseline: 1.0228x; 1.0228x over previous
import functools

import jax
import jax.numpy as jnp
from jax import lax
from jax.experimental import pallas as pl
from jax.experimental.pallas import tpu as pltpu
from jax.experimental.pallas import tpu_sc as plsc

_VOCAB = 1000000
_DIM = 64
_BATCH = 4096
_HIST = 50
_N = _BATCH * _HIST
_NW = 32
_PER_W = _N // _NW
_CHUNK = 800
_NCHUNK = _PER_W // _CHUNK


def _make_sc_kernel():
    mesh = plsc.VectorSubcoreMesh(core_axis_name="c", subcore_axis_name="s")

    @functools.partial(
        pl.kernel,
        mesh=mesh,
        compiler_params=pltpu.CompilerParams(use_tc_tiling_on_sc=False),
        out_type=jax.ShapeDtypeStruct((_N, _DIM), jnp.float32),
        scratch_types=[
            pltpu.VMEM((_CHUNK,), jnp.int32),
            pltpu.VMEM((_CHUNK,), jnp.int32),
            pltpu.VMEM((_CHUNK,), jnp.int32),
            pltpu.VMEM((_CHUNK,), jnp.int32),
            pltpu.VMEM((_CHUNK, _DIM), jnp.float32),
            pltpu.VMEM((_CHUNK, _DIM), jnp.float32),
            pltpu.SemaphoreType.DMA,
            pltpu.SemaphoreType.DMA,
            pltpu.SemaphoreType.DMA,
            pltpu.SemaphoreType.DMA,
            pltpu.SemaphoreType.DMA,
            pltpu.SemaphoreType.DMA,
            pltpu.SemaphoreType.DMA,
            pltpu.SemaphoreType.DMA,
        ],
    )
    def tok_emb(x_hbm, table_hbm, idmap_hbm, out_hbm,
                xv0, xv1, mv0, mv1, rows0, rows1,
                sx0, sx1, sm0, sm1, sr0, sr1, so0, so1):
        xv = (xv0, xv1)
        mv = (mv0, mv1)
        rows = (rows0, rows1)
        sx = (sx0, sx1)
        sm = (sm0, sm1)
        sr = (sr0, sr1)
        so = (so0, so1)
        wid = lax.axis_index("s") * 2 + lax.axis_index("c")
        base = wid * _PER_W

        def x_copy(i, b):
            return pltpu.make_async_copy(
                x_hbm.at[pl.ds(base + i * _CHUNK, _CHUNK)], xv[b], sx[b])

        def m_copy(b):
            return pltpu.make_async_copy(idmap_hbm.at[xv[b]], mv[b], sm[b])

        def r_copy(b):
            return pltpu.make_async_copy(table_hbm.at[mv[b]], rows[b], sr[b])

        def o_copy(i, b):
            return pltpu.make_async_copy(
                rows[b], out_hbm.at[pl.ds(base + i * _CHUNK, _CHUNK)], so[b])

        x_copy(0, 0).start()
        x_copy(1, 1).start()
        x_copy(0, 0).wait()
        m_copy(0).start()

        def body(g2, carry):
            g = g2 * 2
            for b in range(2):
                i = g + b
                nb = 1 - b
                m_copy(b).wait()

                @pl.when(i >= 2)
                def _():
                    o_copy(i - 2, b).wait()

                r_copy(b).start()

                @pl.when(i + 1 < _NCHUNK)
                def _():
                    x_copy(i + 1, nb).wait()
                    m_copy(nb).start()

                @pl.when(i + 2 < _NCHUNK)
                def _():
                    x_copy(i + 2, b).start()

                r_copy(b).wait()
                o_copy(i, b).start()
            return carry

        lax.fori_loop(0, _NCHUNK // 2, body, 0)

        o_copy(_NCHUNK - 2, 0).wait()
        o_copy(_NCHUNK - 1, 1).wait()

    return tok_emb


_SC_KERNEL = _make_sc_kernel()


def kernel(x, table, id_mapper):
    x_flat = x.reshape(_N)
    out = _SC_KERNEL(x_flat, table, id_mapper)
    return out.reshape(_BATCH, _HIST, _DIM)


# speedup vs baseline: 1.0242x; 1.0013x over previous
import functools

import jax
import jax.numpy as jnp
from jax import lax
from jax.experimental import pallas as pl
from jax.experimental.pallas import tpu as pltpu
from jax.experimental.pallas import tpu_sc as plsc

_VOCAB = 1000000
_DIM = 64
_BATCH = 4096
_HIST = 50
_N = _BATCH * _HIST
_NW = 32
_PER_W = _N // _NW
_CHUNK = 400
_NCHUNK = _PER_W // _CHUNK
_NBUF = 4


def _make_sc_kernel():
    mesh = plsc.VectorSubcoreMesh(core_axis_name="c", subcore_axis_name="s")

    @functools.partial(
        pl.kernel,
        mesh=mesh,
        compiler_params=pltpu.CompilerParams(use_tc_tiling_on_sc=False),
        out_type=jax.ShapeDtypeStruct((_N, _DIM), jnp.float32),
        scratch_types=[
            pltpu.VMEM((_PER_W,), jnp.int32),
            pltpu.VMEM((_PER_W,), jnp.int32),
            pltpu.VMEM((_NBUF, _CHUNK, _DIM), jnp.float32),
            pltpu.SemaphoreType.DMA,
            pltpu.SemaphoreType.DMA,
        ] + [pltpu.SemaphoreType.DMA] * _NBUF
          + [pltpu.SemaphoreType.DMA] * _NBUF,
    )
    def tok_emb(x_hbm, table_hbm, idmap_hbm, out_hbm,
                xv, mv, rows, sem_x, sem_m, *bufsems):
        sem_r = bufsems[:_NBUF]
        sem_o = bufsems[_NBUF:]
        wid = lax.axis_index("s") * 2 + lax.axis_index("c")
        base = wid * _PER_W

        def r_copy(i, b):
            return pltpu.make_async_copy(
                table_hbm.at[mv.at[pl.ds(i * _CHUNK, _CHUNK)]],
                rows.at[b], sem_r[b])

        def o_copy(i, b):
            return pltpu.make_async_copy(
                rows.at[b], out_hbm.at[pl.ds(base + i * _CHUNK, _CHUNK)],
                sem_o[b])

        pltpu.make_async_copy(
            x_hbm.at[pl.ds(base, _PER_W)], xv, sem_x).start()
        pltpu.make_async_copy(
            x_hbm.at[pl.ds(base, _PER_W)], xv, sem_x).wait()
        pltpu.make_async_copy(idmap_hbm.at[xv], mv, sem_m).start()
        pltpu.make_async_copy(idmap_hbm.at[xv], mv, sem_m).wait()

        for i in range(_NCHUNK + 3):
            if i < _NCHUNK:
                b = i % _NBUF
                if i >= _NBUF:
                    o_copy(i - _NBUF, b).wait()
                r_copy(i, b).start()
            j = i - 3
            if 0 <= j < _NCHUNK:
                bj = j % _NBUF
                r_copy(j, bj).wait()
                o_copy(j, bj).start()

        for j in range(_NCHUNK - _NBUF, _NCHUNK):
            o_copy(j, j % _NBUF).wait()

    return tok_emb


_SC_KERNEL = _make_sc_kernel()


def kernel(x, table, id_mapper):
    x_flat = x.reshape(_N)
    out = _SC_KERNEL(x_flat, table, id_mapper)
    return out.reshape(_BATCH, _HIST, _DIM)
